# baseline (speedup 1.0000x reference)
import functools

import jax
import jax.numpy as jnp
from jax import lax
from jax.experimental import pallas as pl
from jax.experimental.pallas import tpu as pltpu
from jax.experimental.pallas import tpu_sc as plsc

BATCH = 16384
EMBED_DIM = 16
GROUP = 8
NROW = 1000000
NGRP = NROW // GROUP

_info = plsc.get_sparse_core_info()
_NC = _info.num_cores
_NS = _info.num_subcores
_NW = _NC * _NS
_BPW = BATCH // _NW
_CHUNK = 128
_NCHUNK = _BPW // _CHUNK

_NBLK = NROW // 128
_STRIDED = _NBLK - 4
_PAIRS = (_STRIDED // _NW) // 2

_mesh = plsc.VectorSubcoreMesh(core_axis_name="c", subcore_axis_name="s")


@functools.partial(
    pl.kernel,
    out_type=(
        jax.ShapeDtypeStruct((NGRP, 128), jnp.float32),
        jax.ShapeDtypeStruct((NGRP, 128), jnp.float32),
    ),
    mesh=_mesh,
    scratch_types=[
        pltpu.VMEM((2, EMBED_DIM, 128), jnp.float32),
        pltpu.VMEM((2, EMBED_DIM, 128), jnp.float32),
        pltpu.VMEM((2, EMBED_DIM, 128), jnp.float32),
        pltpu.VMEM((2, EMBED_DIM, 128), jnp.float32),
        pltpu.SemaphoreType.DMA,
        pltpu.SemaphoreType.DMA,
        pltpu.SemaphoreType.DMA,
        pltpu.SemaphoreType.DMA,
    ],
    compiler_params=pltpu.CompilerParams(needs_layout_passes=False),
)
def _sc_relayout(utab, itab, utail, itail, uscr, iscr, ubin, ibin, ubout,
                 ibout, sin_u, sin_i, sout_u, sout_i):
    wid = lax.axis_index("s") * _NC + lax.axis_index("c")
    lane = lax.iota(jnp.int32, 16)

    def fire_in(b, par):
        pltpu.async_copy(utab.at[:, pl.ds(b * 128, 128)], ubin.at[par], sin_u)
        pltpu.async_copy(itab.at[:, pl.ds(b * 128, 128)], ibin.at[par], sin_i)

    def wait_in(par):
        pltpu.make_async_copy(utab.at[:, pl.ds(0, 128)], ubin.at[par],
                              sin_u).wait()
        pltpu.make_async_copy(itab.at[:, pl.ds(0, 128)], ibin.at[par],
                              sin_i).wait()

    def fire_out(b, par):
        pltpu.async_copy(ubout.at[par], uscr.at[pl.ds(b * 16, 16), :], sout_u)
        pltpu.async_copy(ibout.at[par], iscr.at[pl.ds(b * 16, 16), :], sout_i)

    def wait_out(par):
        pltpu.make_async_copy(uscr.at[pl.ds(0, 16), :], ubout.at[par],
                              sout_u).wait()
        pltpu.make_async_copy(iscr.at[pl.ds(0, 16), :], ibout.at[par],
                              sout_i).wait()

    def transpose_slab(src, dst, par, ngroups):
        pvec = jnp.full((16,), par, jnp.int32)
        for g in range(ngroups):
            for s in range(GROUP):
                col = jnp.full((16,), g * GROUP + s, jnp.int32)
                vals = plsc.load_gather(src, [pvec, lane, col])
                dst[par, g, pl.ds(s * EMBED_DIM, EMBED_DIM)] = vals

    fire_in(wid, 0)
    fire_in(wid + _NW, 1)

    def body(kk, _):
        for par in range(2):
            b = wid + _NW * (2 * kk + par)
            wait_in(par)

            @pl.when(kk > 0)
            def _():
                wait_out(par)

            transpose_slab(ubin, ubout, par, 16)
            transpose_slab(ibin, ibout, par, 16)
            fire_out(b, par)

            @pl.when(kk < _PAIRS - 1)
            def _():
                fire_in(b + 2 * _NW, par)
        return ()

    lax.fori_loop(0, _PAIRS, body, (), unroll=False)
    wait_out(0)
    wait_out(1)

    @pl.when(wid < 4)
    def _():
        b = _STRIDED + wid
        pltpu.sync_copy(utab.at[:, pl.ds(b * 128, 128)], ubin.at[0])
        pltpu.sync_copy(itab.at[:, pl.ds(b * 128, 128)], ibin.at[0])
        transpose_slab(ubin, ubout, 0, 16)
        transpose_slab(ibin, ibout, 0, 16)
        pltpu.sync_copy(ubout.at[0], uscr.at[pl.ds(b * 16, 16), :])
        pltpu.sync_copy(ibout.at[0], iscr.at[pl.ds(b * 16, 16), :])

    @pl.when(wid == 4)
    def _():
        pltpu.sync_copy(utail, ubin.at[0, pl.ds(0, 8), :])
        pltpu.sync_copy(itail, ibin.at[0, pl.ds(0, 8), :])
        pltpu.sync_copy(ubin.at[0, pl.ds(0, 8), :],
                        uscr.at[pl.ds(_NBLK * 16, 8), :])
        pltpu.sync_copy(ibin.at[0, pl.ds(0, 8), :],
                        iscr.at[pl.ds(_NBLK * 16, 8), :])


def _extract_chunk(rows_v, idx_v, out_v, c):
    lane = lax.iota(jnp.int32, 16)
    for t in range(_CHUNK // 16):
        idx16 = idx_v[pl.ds(c * _CHUNK + t * 16, 16)]
        rows = lane + t * 16
        cols = (idx16 & 7) * EMBED_DIM
        for d in range(EMBED_DIM):
            vals = plsc.load_gather(rows_v, [rows, cols + d])
            out_v[d, pl.ds(c * _CHUNK + t * 16, 16)] = vals


@functools.partial(
    pl.kernel,
    out_type=(
        jax.ShapeDtypeStruct((EMBED_DIM, BATCH), jnp.float32),
        jax.ShapeDtypeStruct((EMBED_DIM, BATCH), jnp.float32),
    ),
    mesh=_mesh,
    scratch_types=[
        pltpu.VMEM((_BPW,), jnp.int32),
        pltpu.VMEM((_BPW,), jnp.int32),
        pltpu.VMEM((_BPW,), jnp.int32),
        pltpu.VMEM((_BPW,), jnp.int32),
        pltpu.VMEM((2, _CHUNK, 128), jnp.float32),
        pltpu.VMEM((2, _CHUNK, 128), jnp.float32),
        pltpu.VMEM((EMBED_DIM, _BPW), jnp.float32),
        pltpu.VMEM((EMBED_DIM, _BPW), jnp.float32),
        pltpu.SemaphoreType.DMA,
        pltpu.SemaphoreType.DMA,
    ],
    compiler_params=pltpu.CompilerParams(needs_layout_passes=False),
)
def _sc_gather2(uidx_hbm, iidx_hbm, utab_hbm, itab_hbm, uout_hbm, iout_hbm,
                uidx_v, iidx_v, ug_v, ig_v, ubuf, ibuf, uout_v, iout_v,
                usem, isem):
    wid = lax.axis_index("s") * _NC + lax.axis_index("c")
    base = wid * _BPW
    pltpu.sync_copy(uidx_hbm.at[pl.ds(base, _BPW)], uidx_v)
    pltpu.sync_copy(iidx_hbm.at[pl.ds(base, _BPW)], iidx_v)
    for k in range(_BPW // 16):
        s = pl.ds(k * 16, 16)
        ug_v[s] = lax.shift_right_logical(uidx_v[s], 3)
        ig_v[s] = lax.shift_right_logical(iidx_v[s], 3)

    def fire(c):
        b = c % 2
        cu = pltpu.async_copy(utab_hbm.at[ug_v.at[pl.ds(c * _CHUNK, _CHUNK)]],
                              ubuf.at[b], usem)
        ci = pltpu.async_copy(itab_hbm.at[ig_v.at[pl.ds(c * _CHUNK, _CHUNK)]],
                              ibuf.at[b], isem)
        return cu, ci

    inflight = fire(0)
    for c in range(_NCHUNK):
        cu, ci = inflight
        if c + 1 < _NCHUNK:
            inflight = fire(c + 1)
        cu.wait()
        _extract_chunk(ubuf.at[c % 2], uidx_v, uout_v, c)
        ci.wait()
        _extract_chunk(ibuf.at[c % 2], iidx_v, iout_v, c)

    pltpu.sync_copy(uout_v, uout_hbm.at[:, pl.ds(base, _BPW)])
    pltpu.sync_copy(iout_v, iout_hbm.at[:, pl.ds(base, _BPW)])


def _mlp_body(ut_ref, it_ref, w1ut_ref, w1it_ref, b1_ref, w3_ref, b3_ref,
              o_ref):
    h = (jnp.dot(w1ut_ref[...], ut_ref[...],
                 preferred_element_type=jnp.float32)
         + jnp.dot(w1it_ref[...], it_ref[...],
                   preferred_element_type=jnp.float32)
         + b1_ref[...][:, None])
    h = jnp.maximum(h, 0.0)
    o = jnp.sum(h * w3_ref[...][:, None], axis=0) + b3_ref[...]
    o_ref[...] = jax.nn.sigmoid(o)


def _tc_mlp(ut, it, w1ut, w1it, b1, w3, b3):
    return pl.pallas_call(
        _mlp_body,
        out_shape=jax.ShapeDtypeStruct((BATCH,), jnp.float32),
    )(ut, it, w1ut, w1it, b1, w3, b3)


def kernel(user_indices, item_indices, emb_user, emb_item, W1, b1, W3, b3):
    uidx = user_indices.astype(jnp.int32)
    iidx = item_indices.astype(jnp.int32)
    utail = emb_user[NROW - 64:].reshape(8, 128)
    itail = emb_item[NROW - 64:].reshape(8, 128)
    uscr, iscr = _sc_relayout(emb_user.T, emb_item.T, utail, itail)
    u_t, i_t = _sc_gather2(uidx, iidx, uscr, iscr)
    w1ut = W1[:EMBED_DIM].T
    w1it = W1[EMBED_DIM:].T
    w3 = W3[:, 0]
    return _tc_mlp(u_t, i_t, w1ut, w1it, b1, w3, b3)


# speedup vs baseline: 1.2214x; 1.2214x over previous
import functools

import jax
import jax.numpy as jnp
from jax import lax
from jax.experimental import pallas as pl
from jax.experimental.pallas import tpu as pltpu
from jax.experimental.pallas import tpu_sc as plsc

BATCH = 16384
EMBED_DIM = 16
GROUP = 8
NROW = 1000000
NGRP = NROW // GROUP

_info = plsc.get_sparse_core_info()
_NC = _info.num_cores
_NS = _info.num_subcores
_NW = _NC * _NS
_BPW = BATCH // _NW
_CHUNK = 128
_NCHUNK = _BPW // _CHUNK

_NBLK = NROW // 128
_STRIDED = _NBLK - 4
_RING = 4
_ROUNDS = (_STRIDED // _NW) // _RING

_mesh = plsc.VectorSubcoreMesh(core_axis_name="c", subcore_axis_name="s")


@functools.partial(
    pl.kernel,
    out_type=(
        jax.ShapeDtypeStruct((NGRP, 128), jnp.float32),
        jax.ShapeDtypeStruct((NGRP, 128), jnp.float32),
    ),
    mesh=_mesh,
    scratch_types=[
        pltpu.VMEM((_RING, EMBED_DIM, 128), jnp.float32),
        pltpu.VMEM((_RING, EMBED_DIM, 128), jnp.float32),
        pltpu.VMEM((_RING, EMBED_DIM, 128), jnp.float32),
        pltpu.VMEM((_RING, EMBED_DIM, 128), jnp.float32),
        pltpu.SemaphoreType.DMA,
        pltpu.SemaphoreType.DMA,
        pltpu.SemaphoreType.DMA,
        pltpu.SemaphoreType.DMA,
    ],
    compiler_params=pltpu.CompilerParams(needs_layout_passes=False),
)
def _sc_relayout(utab, itab, utail, itail, uscr, iscr, ubin, ibin, ubout,
                 ibout, sin_u, sin_i, sout_u, sout_i):
    wid = lax.axis_index("s") * _NC + lax.axis_index("c")
    lane = lax.iota(jnp.int32, 16)

    def fire_in(b, q):
        pltpu.async_copy(utab.at[:, pl.ds(b * 128, 128)], ubin.at[q], sin_u)
        pltpu.async_copy(itab.at[:, pl.ds(b * 128, 128)], ibin.at[q], sin_i)

    def wait_in(q):
        pltpu.make_async_copy(utab.at[:, pl.ds(0, 128)], ubin.at[q],
                              sin_u).wait()
        pltpu.make_async_copy(itab.at[:, pl.ds(0, 128)], ibin.at[q],
                              sin_i).wait()

    def fire_out(b, q):
        pltpu.async_copy(ubout.at[q], uscr.at[pl.ds(b * 16, 16), :], sout_u)
        pltpu.async_copy(ibout.at[q], iscr.at[pl.ds(b * 16, 16), :], sout_i)

    def wait_out(q):
        pltpu.make_async_copy(uscr.at[pl.ds(0, 16), :], ubout.at[q],
                              sout_u).wait()
        pltpu.make_async_copy(iscr.at[pl.ds(0, 16), :], ibout.at[q],
                              sout_i).wait()

    def transpose_slabs(q, ngroups=16):
        qvec = jnp.full((16,), q, jnp.int32)
        for g in range(ngroups):
            for s in range(GROUP):
                col = jnp.full((16,), g * GROUP + s, jnp.int32)
                uv = plsc.load_gather(ubin, [qvec, lane, col])
                iv = plsc.load_gather(ibin, [qvec, lane, col])
                ubout[q, g, pl.ds(s * EMBED_DIM, EMBED_DIM)] = uv
                ibout[q, g, pl.ds(s * EMBED_DIM, EMBED_DIM)] = iv

    for q in range(_RING):
        fire_in(wid + _NW * q, q)

    def body(k, _):
        for q in range(_RING):
            b = wid + _NW * (_RING * k + q)
            wait_in(q)

            @pl.when(k > 0)
            def _():
                wait_out(q)

            transpose_slabs(q)
            fire_out(b, q)

            @pl.when(k < _ROUNDS - 1)
            def _():
                fire_in(b + _NW * _RING, q)
        return ()

    lax.fori_loop(0, _ROUNDS, body, (), unroll=False)
    for q in range(_RING):
        wait_out(q)

    @pl.when(wid < 4)
    def _():
        b = _STRIDED + wid
        pltpu.sync_copy(utab.at[:, pl.ds(b * 128, 128)], ubin.at[0])
        pltpu.sync_copy(itab.at[:, pl.ds(b * 128, 128)], ibin.at[0])
        transpose_slabs(0)
        pltpu.sync_copy(ubout.at[0], uscr.at[pl.ds(b * 16, 16), :])
        pltpu.sync_copy(ibout.at[0], iscr.at[pl.ds(b * 16, 16), :])

    @pl.when(wid == 4)
    def _():
        pltpu.sync_copy(utail, ubin.at[0, pl.ds(0, 8), :])
        pltpu.sync_copy(itail, ibin.at[0, pl.ds(0, 8), :])
        pltpu.sync_copy(ubin.at[0, pl.ds(0, 8), :],
                        uscr.at[pl.ds(_NBLK * 16, 8), :])
        pltpu.sync_copy(ibin.at[0, pl.ds(0, 8), :],
                        iscr.at[pl.ds(_NBLK * 16, 8), :])


def _extract_chunk(rows_v, idx_v, out_v, c):
    lane = lax.iota(jnp.int32, 16)
    for t in range(_CHUNK // 16):
        idx16 = idx_v[pl.ds(c * _CHUNK + t * 16, 16)]
        rows = lane + t * 16
        cols = (idx16 & 7) * EMBED_DIM
        for d in range(EMBED_DIM):
            vals = plsc.load_gather(rows_v, [rows, cols + d])
            out_v[d, pl.ds(c * _CHUNK + t * 16, 16)] = vals


@functools.partial(
    pl.kernel,
    out_type=(
        jax.ShapeDtypeStruct((EMBED_DIM, BATCH), jnp.float32),
        jax.ShapeDtypeStruct((EMBED_DIM, BATCH), jnp.float32),
    ),
    mesh=_mesh,
    scratch_types=[
        pltpu.VMEM((_BPW,), jnp.int32),
        pltpu.VMEM((_BPW,), jnp.int32),
        pltpu.VMEM((_BPW,), jnp.int32),
        pltpu.VMEM((_BPW,), jnp.int32),
        pltpu.VMEM((2, _CHUNK, 128), jnp.float32),
        pltpu.VMEM((2, _CHUNK, 128), jnp.float32),
        pltpu.VMEM((EMBED_DIM, _BPW), jnp.float32),
        pltpu.VMEM((EMBED_DIM, _BPW), jnp.float32),
        pltpu.SemaphoreType.DMA,
        pltpu.SemaphoreType.DMA,
    ],
    compiler_params=pltpu.CompilerParams(needs_layout_passes=False),
)
def _sc_gather2(uidx_hbm, iidx_hbm, utab_hbm, itab_hbm, uout_hbm, iout_hbm,
                uidx_v, iidx_v, ug_v, ig_v, ubuf, ibuf, uout_v, iout_v,
                usem, isem):
    wid = lax.axis_index("s") * _NC + lax.axis_index("c")
    base = wid * _BPW
    pltpu.sync_copy(uidx_hbm.at[pl.ds(base, _BPW)], uidx_v)
    pltpu.sync_copy(iidx_hbm.at[pl.ds(base, _BPW)], iidx_v)
    for k in range(_BPW // 16):
        s = pl.ds(k * 16, 16)
        ug_v[s] = lax.shift_right_logical(uidx_v[s], 3)
        ig_v[s] = lax.shift_right_logical(iidx_v[s], 3)

    def fire(c):
        b = c % 2
        cu = pltpu.async_copy(utab_hbm.at[ug_v.at[pl.ds(c * _CHUNK, _CHUNK)]],
                              ubuf.at[b], usem)
        ci = pltpu.async_copy(itab_hbm.at[ig_v.at[pl.ds(c * _CHUNK, _CHUNK)]],
                              ibuf.at[b], isem)
        return cu, ci

    inflight = fire(0)
    for c in range(_NCHUNK):
        cu, ci = inflight
        if c + 1 < _NCHUNK:
            inflight = fire(c + 1)
        cu.wait()
        _extract_chunk(ubuf.at[c % 2], uidx_v, uout_v, c)
        ci.wait()
        _extract_chunk(ibuf.at[c % 2], iidx_v, iout_v, c)

    pltpu.sync_copy(uout_v, uout_hbm.at[:, pl.ds(base, _BPW)])
    pltpu.sync_copy(iout_v, iout_hbm.at[:, pl.ds(base, _BPW)])


def _mlp_body(ut_ref, it_ref, w1ut_ref, w1it_ref, b1_ref, w3_ref, b3_ref,
              o_ref):
    h = (jnp.dot(w1ut_ref[...], ut_ref[...],
                 preferred_element_type=jnp.float32)
         + jnp.dot(w1it_ref[...], it_ref[...],
                   preferred_element_type=jnp.float32)
         + b1_ref[...][:, None])
    h = jnp.maximum(h, 0.0)
    o = jnp.sum(h * w3_ref[...][:, None], axis=0) + b3_ref[...]
    o_ref[...] = jax.nn.sigmoid(o)


def _tc_mlp(ut, it, w1ut, w1it, b1, w3, b3):
    return pl.pallas_call(
        _mlp_body,
        out_shape=jax.ShapeDtypeStruct((BATCH,), jnp.float32),
    )(ut, it, w1ut, w1it, b1, w3, b3)


def kernel(user_indices, item_indices, emb_user, emb_item, W1, b1, W3, b3):
    uidx = user_indices.astype(jnp.int32)
    iidx = item_indices.astype(jnp.int32)
    utail = emb_user[NROW - 64:].reshape(8, 128)
    itail = emb_item[NROW - 64:].reshape(8, 128)
    uscr, iscr = _sc_relayout(emb_user.T, emb_item.T, utail, itail)
    u_t, i_t = _sc_gather2(uidx, iidx, uscr, iscr)
    w1ut = W1[:EMBED_DIM].T
    w1it = W1[EMBED_DIM:].T
    w3 = W3[:, 0]
    return _tc_mlp(u_t, i_t, w1ut, w1it, b1, w3, b3)


# speedup vs baseline: 7.8207x; 6.4032x over previous
import functools

import jax
import jax.numpy as jnp
from jax import lax
from jax.experimental import pallas as pl
from jax.experimental.pallas import tpu as pltpu
from jax.experimental.pallas import tpu_sc as plsc

BATCH = 16384
EMBED_DIM = 16
NROW = 1000000

_info = plsc.get_sparse_core_info()
_NC = _info.num_cores
_NS = _info.num_subcores
_NW = _NC * _NS
_BPW = BATCH // _NW
_K = 16
_NCH = _BPW // _K

_mesh = plsc.VectorSubcoreMesh(core_axis_name="c", subcore_axis_name="s")


@functools.partial(
    pl.kernel,
    out_type=(
        jax.ShapeDtypeStruct((EMBED_DIM, BATCH), jnp.float32),
        jax.ShapeDtypeStruct((EMBED_DIM, BATCH), jnp.float32),
    ),
    mesh=_mesh,
    scratch_types=[
        pltpu.VMEM((_BPW,), jnp.int32),
        pltpu.VMEM((_BPW,), jnp.int32),
        pltpu.VMEM((_K, EMBED_DIM, 128), jnp.float32),
        pltpu.VMEM((_K, EMBED_DIM, 128), jnp.float32),
        pltpu.VMEM((EMBED_DIM, _BPW), jnp.float32),
        pltpu.VMEM((EMBED_DIM, _BPW), jnp.float32),
        pltpu.SemaphoreType.DMA,
        pltpu.SemaphoreType.DMA,
    ],
    compiler_params=pltpu.CompilerParams(needs_layout_passes=False),
)
def _sc_gather(uidx_hbm, iidx_hbm, utab, itab, uout_hbm, iout_hbm,
               uidx_v, iidx_v, uring, iring, uout_v, iout_v, usem, isem):
    wid = lax.axis_index("s") * _NC + lax.axis_index("c")
    base = wid * _BPW
    pltpu.sync_copy(uidx_hbm.at[pl.ds(base, _BPW)], uidx_v)
    pltpu.sync_copy(iidx_hbm.at[pl.ds(base, _BPW)], iidx_v)

    lane = lax.iota(jnp.int32, 16)
    lane128 = lane * 128

    def fire(kk, q):
        j = kk * _K + q
        uvec = uidx_v[pl.ds(kk * _K, _K)]
        ivec = iidx_v[pl.ds(kk * _K, _K)]
        ub = pl.multiple_of((uvec[q] >> 7) * 128, 128)
        ib = pl.multiple_of((ivec[q] >> 7) * 128, 128)
        pltpu.async_copy(utab.at[:, pl.ds(ub, 128)], uring.at[q], usem)
        pltpu.async_copy(itab.at[:, pl.ds(ib, 128)], iring.at[q], isem)
        return j

    def wait(ring, sem, q):
        pltpu.make_async_copy(utab.at[:, pl.ds(0, 128)], ring.at[q],
                              sem).wait()

    for q in range(_K):
        fire(0, q)

    def body(kk, _):
        uvec = uidx_v[pl.ds(kk * _K, _K)]
        ivec = iidx_v[pl.ds(kk * _K, _K)]
        ucols = uvec & 127
        icols = ivec & 127
        jbase = jnp.full((16,), kk * _K, jnp.int32)
        for q in range(_K):
            jvec = jbase + q
            wait(uring, usem, q)
            uc = jnp.full((16,), 1, jnp.int32) * ucols[q]
            uv = plsc.load_gather(uring, [jnp.full((16,), q, jnp.int32),
                                          lane, uc])
            plsc.store_scatter(uout_v, [lane, jvec], uv)
            wait(iring, isem, q)
            ic = jnp.full((16,), 1, jnp.int32) * icols[q]
            iv = plsc.load_gather(iring, [jnp.full((16,), q, jnp.int32),
                                          lane, ic])
            plsc.store_scatter(iout_v, [lane, jvec], iv)

            @pl.when(kk < _NCH - 1)
            def _():
                fire(kk + 1, q)
        return ()

    lax.fori_loop(0, _NCH, body, (), unroll=False)

    pltpu.sync_copy(uout_v, uout_hbm.at[:, pl.ds(base, _BPW)])
    pltpu.sync_copy(iout_v, iout_hbm.at[:, pl.ds(base, _BPW)])


def _mlp_body(ut_ref, it_ref, w1ut_ref, w1it_ref, b1_ref, w3_ref, b3_ref,
              o_ref):
    h = (jnp.dot(w1ut_ref[...], ut_ref[...],
                 preferred_element_type=jnp.float32)
         + jnp.dot(w1it_ref[...], it_ref[...],
                   preferred_element_type=jnp.float32)
         + b1_ref[...][:, None])
    h = jnp.maximum(h, 0.0)
    o = jnp.sum(h * w3_ref[...][:, None], axis=0) + b3_ref[...]
    o_ref[...] = jax.nn.sigmoid(o)


def _tc_mlp(ut, it, w1ut, w1it, b1, w3, b3):
    return pl.pallas_call(
        _mlp_body,
        out_shape=jax.ShapeDtypeStruct((BATCH,), jnp.float32),
    )(ut, it, w1ut, w1it, b1, w3, b3)


def kernel(user_indices, item_indices, emb_user, emb_item, W1, b1, W3, b3):
    uidx = user_indices.astype(jnp.int32)
    iidx = item_indices.astype(jnp.int32)
    u_t, i_t = _sc_gather(uidx, iidx, emb_user.T, emb_item.T)
    w1ut = W1[:EMBED_DIM].T
    w1it = W1[EMBED_DIM:].T
    w3 = W3[:, 0]
    return _tc_mlp(u_t, i_t, w1ut, w1it, b1, w3, b3)
